# baseline (speedup 1.0000x reference)
import functools

import jax
import jax.numpy as jnp
from jax import lax
from jax.experimental import pallas as pl
from jax.experimental.pallas import tpu as pltpu
from jax.experimental.pallas import tpu_sc as plsc

VOCAB = 201
HIDDEN = 1024
OUT = 41
BATCH = 4
SEQ = 4096
B_TOK = BATCH * SEQ

_NC, _NS = 2, 16
_NW = _NC * _NS
_BPW = B_TOK // _NW
_WPB = SEQ // _BPW


def _proj_body(emb_ref, w_ref, b_ref, p_ref):
    p_ref[...] = (
        lax.dot_general(
            emb_ref[...], w_ref[...],
            (((1,), (1,)), ((), ())),
            preferred_element_type=jnp.float32,
        )
        + b_ref[...]
    )


def _project_table(emb_table, W, b2d):
    return pl.pallas_call(
        _proj_body,
        out_shape=jax.ShapeDtypeStruct((VOCAB, OUT), jnp.float32),
    )(emb_table, W, b2d)


@functools.cache
def _make_sc_gather():
    mesh = plsc.VectorSubcoreMesh(core_axis_name="c", subcore_axis_name="s")

    @functools.partial(
        pl.kernel,
        out_type=jax.ShapeDtypeStruct((BATCH, SEQ, OUT), jnp.float32),
        mesh=mesh,
        scratch_types=[
            pltpu.VMEM((VOCAB, OUT), jnp.float32),
            pltpu.VMEM((_BPW,), jnp.int32),
            pltpu.VMEM((_BPW, OUT), jnp.float32),
        ],
        compiler_params=pltpu.CompilerParams(needs_layout_passes=False),
    )
    def _sc_gather(table_hbm, idx_hbm, out_hbm, table_v, idx_v, rows_v):
        wid = lax.axis_index("s") * _NC + lax.axis_index("c")
        bi = wid // _WPB
        si = wid % _WPB
        pltpu.sync_copy(table_hbm, table_v)
        pltpu.sync_copy(idx_hbm.at[bi, pl.ds(si * _BPW, _BPW)], idx_v)
        lanes = lax.iota(jnp.int32, 16)
        coffs = [off + lanes for off in (0, 16, OUT - 16)]

        @plsc.parallel_loop(0, _BPW // 16, 1, unroll=1)
        def _(g):
            base = pl.multiple_of(g * 16, 16)
            rbase = idx_v[pl.ds(base, 16)]
            for l in range(16):
                rs = lax.gather(
                    rbase, jnp.full((16, 1), l, jnp.int32),
                    lax.GatherDimensionNumbers((), (0,), (0,)), (1,),
                    mode=lax.GatherScatterMode.PROMISE_IN_BOUNDS)
                t = base + l
                for off, coff in zip((0, 16, OUT - 16), coffs):
                    vals = plsc.load_gather(table_v, [rs, coff])
                    rows_v[t, pl.ds(off, 16)] = vals

        pltpu.sync_copy(rows_v, out_hbm.at[bi, pl.ds(si * _BPW, _BPW), :])

    return _sc_gather


def kernel(x, emb_table, W, b):
    table = _project_table(emb_table, W, b.reshape(1, OUT))
    return _make_sc_gather()(table, x)


# speedup vs baseline: 1.3584x; 1.3584x over previous
import functools

import jax
import jax.numpy as jnp
from jax import lax
from jax.experimental import pallas as pl
from jax.experimental.pallas import tpu as pltpu
from jax.experimental.pallas import tpu_sc as plsc

VOCAB = 201
HIDDEN = 1024
OUT = 41
BATCH = 4
SEQ = 4096
B_TOK = BATCH * SEQ

_NC, _NS = 2, 16
_NW = _NC * _NS
_BPW = B_TOK // _NW
_WPB = SEQ // _BPW
_NJ = _BPW // 128


def _proj_body(emb_ref, w_ref, b_ref, p_ref):
    p_ref[...] = (
        lax.dot_general(
            emb_ref[...], w_ref[...],
            (((1,), (1,)), ((), ())),
            preferred_element_type=jnp.float32,
        )
        + b_ref[...]
    )


def _project_table(emb_table, W, b2d):
    return pl.pallas_call(
        _proj_body,
        out_shape=jax.ShapeDtypeStruct((VOCAB, OUT), jnp.float32),
    )(emb_table, W, b2d)


@functools.cache
def _make_sc_gather():
    mesh = plsc.VectorSubcoreMesh(core_axis_name="c", subcore_axis_name="s")

    @functools.partial(
        pl.kernel,
        out_type=jax.ShapeDtypeStruct((OUT, BATCH * 32, 128), jnp.float32),
        mesh=mesh,
        scratch_types=[
            pltpu.VMEM((VOCAB * OUT,), jnp.float32),
            pltpu.VMEM((_BPW,), jnp.int32),
            pltpu.VMEM((_NJ, OUT, 1, 128), jnp.float32),
            pltpu.SemaphoreType.DMA,
        ],
        compiler_params=pltpu.CompilerParams(needs_layout_passes=False),
    )
    def _sc_gather(table_hbm, idx_hbm, out_hbm, table_v, idx_v, rows_v, sem):
        wid = lax.axis_index("s") * _NC + lax.axis_index("c")
        bi = wid // _WPB
        si = wid % _WPB
        pltpu.sync_copy(table_hbm, table_v)
        pltpu.sync_copy(idx_hbm.at[bi, pl.ds(si * _BPW, _BPW)], idx_v)

        @plsc.parallel_loop(0, _BPW // 16, 1, unroll=2)
        def _(g):
            base = pl.multiple_of(g * 16, 16)
            j = g // 8
            sl = (g % 8) * 16
            rbase = idx_v[pl.ds(base, 16)] * OUT
            for c in range(OUT):
                vals = plsc.load_gather(table_v, [rbase + c])
                rows_v[j, c, 0, pl.ds(sl, 16)] = vals

        copies = []
        for j in range(_NJ):
            row = si * (_NJ * BATCH) + j * BATCH + bi
            copies.append(
                pltpu.async_copy(
                    rows_v.at[j], out_hbm.at[:, pl.ds(row, 1), :], sem))
        for cpy in copies:
            cpy.wait()

    return _sc_gather


def kernel(x, emb_table, W, b):
    table = _project_table(emb_table, W, b.reshape(1, OUT))
    out3 = _make_sc_gather()(table.reshape(VOCAB * OUT), x)
    return (
        out3.reshape(OUT, 32, BATCH, 128)
        .transpose(2, 1, 3, 0)
        .reshape(BATCH, SEQ, OUT)
    )
